# baseline (speedup 1.0000x reference)
import jax
import jax.numpy as jnp
from jax import lax
from jax.experimental import pallas as pl
from jax.experimental.pallas import tpu as pltpu
from jax.experimental.pallas import tpu_sc as plsc

RESO = 96
_R2 = RESO * RESO
_PAIR_J = RESO
_PAIR_I = 97 * RESO
N_RENDER = 262144
N_PIX = 16384
S_RAY = 16
NC, NS, L = 2, 16, 16
NW = NC * NS
PPW = N_RENDER // NW
RPW = N_PIX // NW
CH = 256
NCH = PPW // CH
NG = CH // L
_INV_R2 = 1.0 / float(_R2)
_INV_R = 1.0 / float(RESO)


def _rsqrt(x):
    xi = lax.bitcast_convert_type(x, jnp.int32)
    y = lax.bitcast_convert_type(jnp.int32(0x5F3759DF) - (xi >> 1), jnp.float32)
    h = 0.5 * x
    for _ in range(3):
        y = y * (1.5 - h * y * y)
    return y


def _decode(idxv):
    f = idxv.astype(jnp.float32)
    ii = ((f + 0.5) * _INV_R2).astype(jnp.int32)
    t = idxv - ii * _R2
    jj = ((t.astype(jnp.float32) + 0.5) * _INV_R).astype(jnp.int32)
    kk = t - jj * RESO
    return ii, jj, kk


def _sc_body(table, rpl, vdl, ridx_r, spl, sidx_r, lay, offb,
             outr_h, outg_h, outb_h,
             lay_v, offb_v, p1_v, v1_v, idxt_v,
             idx4_v, rows_v,
             alpha_v, rr_v, rg_v, rb_v,
             outr_v, outg_v, outb_v, semA, semB):
    wid = lax.axis_index("s") * NC + lax.axis_index("c")
    rbase = wid * RPW
    pb = wid * PPW

    pltpu.sync_copy(lay, lay_v)
    pltpu.sync_copy(offb, offb_v)
    pltpu.sync_copy(rpl.at[pl.ds(pb * 3, PPW * 3)], p1_v)
    pltpu.sync_copy(vdl.at[pl.ds(pb * 3, PPW * 3)], v1_v)
    pltpu.sync_copy(ridx_r.at[wid], idxt_v)
    off0 = offb_v[0]
    off1 = offb_v[1]
    off2 = offb_v[2]
    off3 = offb_v[3]
    zero16 = jnp.zeros((L,), jnp.int32)
    one16 = jnp.full((L,), 1, jnp.int32)
    two16 = jnp.full((L,), 2, jnp.int32)
    lanes = lax.iota(jnp.int32, L)
    sems = (semA, semB)

    def fire(c, buf):
        for h in range(2):
            def gidx(g2, c2, _h=h):
                idxv = idxt_v[pl.ds(c * CH + _h * 128 + g2 * L, L)]
                ii, jj, kk = _decode(idxv)
                p0 = (ii * 97 + jj) * RESO + kk
                for r, off in enumerate(
                        (0, _PAIR_J, _PAIR_I, _PAIR_I + _PAIR_J)):
                    idx4_v[buf, 2 * r + _h, pl.ds(g2 * L, L)] = p0 + off
                return c2
            lax.fori_loop(0, NG // 2, gidx, 0)
        for r in range(4):
            for h in range(2):
                pltpu.async_copy(
                    table.at[idx4_v.at[buf, 2 * r + h]],
                    rows_v.at[buf, pl.ds(r * CH + h * 128, 128)], sems[buf])

    def wait_rows(buf):
        pltpu.make_async_copy(table.at[pl.ds(0, 4 * CH)],
                              rows_v.at[buf], sems[buf]).wait()

    def compute(c, buf):
        def gcomp(g, c2):
            o = c * CH + g * L
            pt = g * L + lanes
            idxv = idxt_v[pl.ds(o, L)]
            ii, jj, kk = _decode(idxv)
            ov3 = (o + lanes) * 3
            xq = plsc.load_gather(p1_v, [ov3])
            yq = plsc.load_gather(p1_v, [ov3 + 1])
            zq = plsc.load_gather(p1_v, [ov3 + 2])
            fxv = jnp.clip(xq - ii.astype(jnp.float32), 0.0, 1.0)
            fyv = jnp.clip(yq - jj.astype(jnp.float32), 0.0, 1.0)
            fzv = jnp.clip(zq - kk.astype(jnp.float32), 0.0, 1.0)

            rv = rows_v.at[buf]
            vals = []
            for ch in range(8):
                cz = []
                for q in range(4):
                    aa = plsc.load_gather(
                        rv, [pt + q * CH, jnp.full((L,), ch, jnp.int32)])
                    bb = plsc.load_gather(
                        rv, [pt + q * CH, jnp.full((L,), ch + 8, jnp.int32)])
                    cz.append(aa + (bb - aa) * fzv)
                cy0 = cz[0] + (cz[1] - cz[0]) * fyv
                cy1 = cz[2] + (cz[3] - cz[2]) * fyv
                vals.append(cy0 + (cy1 - cy0) * fxv)

            ax = plsc.load_gather(lay_v, [zero16, ii])
            ay = plsc.load_gather(lay_v, [one16, jj])
            az = plsc.load_gather(lay_v, [two16, kk])
            gxv = 2.0 * ax * xq + off0
            gyv = 2.0 * ay * yq + off1
            gzv = 2.0 * az * zq + off2
            gg = gxv * gxv + gyv * gyv + gzv * gzv
            ninv = 1.0 / (gg * _rsqrt(gg) + 1e-8)
            nx = gxv * ninv
            ny = gyv * ninv
            nz = gzv * ninv

            wx = plsc.load_gather(v1_v, [ov3])
            wy = plsc.load_gather(v1_v, [ov3 + 1])
            wz = plsc.load_gather(v1_v, [ov3 + 2])
            vgg = wx * wx + wy * wy + wz * wz
            vinv = 1.0 / (vgg * _rsqrt(vgg) + 1e-8)
            ux = wx * vinv
            uy = wy * vinv
            uz = wz * vinv

            ndv = jnp.maximum(0.0, -(nx * ux + ny * uy + nz * uz))
            rx = ux + 2.0 * ndv * nx
            ry = uy + 2.0 * ndv * ny
            rz = uz + 2.0 * ndv * nz
            sdot = jnp.maximum(0.0, -(rx * ux + ry * uy + rz * uz))
            s2 = sdot * sdot
            s8 = (s2 * s2) * (s2 * s2)
            sk = s8 * vals[6]
            amb = 0.1 * vals[7]
            dst = pl.ds(o, L)
            rr_v[dst] = vals[0] * ndv + vals[3] * sk + amb * vals[0]
            rg_v[dst] = vals[1] * ndv + vals[4] * sk + amb * vals[1]
            rb_v[dst] = vals[2] * ndv + vals[5] * sk + amb * vals[2]
            return c2
        lax.fori_loop(0, NG, gcomp, 0)

    fire(0, 0)

    def render_pair(p, carry):
        c0 = 2 * p
        fire(c0 + 1, 1)
        wait_rows(0)
        compute(c0, 0)

        @pl.when(p < NCH // 2 - 1)
        def _():
            fire(c0 + 2, 0)
        wait_rows(1)
        compute(c0 + 1, 1)
        return carry
    lax.fori_loop(0, NCH // 2, render_pair, 0)

    pltpu.sync_copy(spl.at[pl.ds(pb * 3, PPW * 3)], p1_v)
    pltpu.sync_copy(sidx_r.at[wid], idxt_v)

    def sdf_group(g, carry):
        o = g * L
        idxv = idxt_v[pl.ds(o, L)]
        ii, jj, kk = _decode(idxv)
        ax = plsc.load_gather(lay_v, [zero16, ii])
        ay = plsc.load_gather(lay_v, [one16, jj])
        az = plsc.load_gather(lay_v, [two16, kk])
        ov3 = (o + lanes) * 3
        xq = plsc.load_gather(p1_v, [ov3])
        yq = plsc.load_gather(p1_v, [ov3 + 1])
        zq = plsc.load_gather(p1_v, [ov3 + 2])
        sdfv = (ax * xq * xq + ay * yq * yq + az * zq * zq
                + off0 * xq + off1 * yq + off2 * zq + off3) / float(RESO)
        alpha_v[pl.ds(o, L)] = 1.0 / (1.0 + jnp.exp(5.0 * sdfv))
        return carry
    lax.fori_loop(0, PPW // L, sdf_group, 0)

    def ray_chunk(rc, carry):
        lidx = lanes * S_RAY + rc * (L * S_RAY)

        def step(s, st):
            T, aR, aG, aB = st
            idx = lidx + s
            av = plsc.load_gather(alpha_v, [idx])
            rv = plsc.load_gather(rr_v, [idx])
            gv = plsc.load_gather(rg_v, [idx])
            bv = plsc.load_gather(rb_v, [idx])
            w = av * T
            return (T * (1.0 - av + 1e-10),
                    aR + w * rv, aG + w * gv, aB + w * bv)
        T0 = jnp.ones((L,), jnp.float32)
        Z0 = jnp.zeros((L,), jnp.float32)
        _, aR, aG, aB = lax.fori_loop(0, S_RAY, step, (T0, Z0, Z0, Z0))
        outr_v[pl.ds(rc * L, L)] = aR
        outg_v[pl.ds(rc * L, L)] = aG
        outb_v[pl.ds(rc * L, L)] = aB
        return carry
    lax.fori_loop(0, RPW // L, ray_chunk, 0)

    pltpu.sync_copy(outr_v, outr_h.at[pl.ds(rbase, RPW)])
    pltpu.sync_copy(outg_v, outg_h.at[pl.ds(rbase, RPW)])
    pltpu.sync_copy(outb_v, outb_h.at[pl.ds(rbase, RPW)])


def kernel(renderPointList, renderIndexList, sdfPointList, sdfIndexList,
           viewDirList, rayList, xLayer, yLayer, zLayer, offset, renderData):
    del rayList
    f32 = jnp.float32
    compact = jnp.concatenate(
        [renderData[..., 0:3], renderData[..., 27:32]], axis=-1)
    pairs = jnp.concatenate(
        [compact[:, :, :RESO, :], compact[:, :, 1:, :]], axis=-1)
    table = pairs.reshape(97 * 97 * RESO, 16)

    ridx_r = renderIndexList.astype(jnp.int32).reshape(NW, PPW)
    sidx_r = sdfIndexList.astype(jnp.int32).reshape(NW, PPW)
    rpl = renderPointList.astype(f32).reshape(-1)
    vdl = viewDirList.astype(f32).reshape(-1)
    spl = sdfPointList.astype(f32).reshape(-1)
    lay = jnp.stack([xLayer, yLayer, zLayer], axis=0).astype(f32)
    offb = jnp.broadcast_to(offset.astype(f32)[:, None], (4, L))

    mesh = plsc.VectorSubcoreMesh(core_axis_name="c", subcore_axis_name="s")
    out_type = (jax.ShapeDtypeStruct((N_PIX,), f32),) * 3
    scratch = [
        pltpu.VMEM((3, RESO), f32),
        pltpu.VMEM((4, L), f32),
        pltpu.VMEM((3 * PPW,), f32),
        pltpu.VMEM((3 * PPW,), f32),
        pltpu.VMEM((PPW,), jnp.int32),
        pltpu.VMEM((2, 8, 128), jnp.int32),
        pltpu.VMEM((2, 4 * CH, 16), f32),
        pltpu.VMEM((PPW,), f32),
        pltpu.VMEM((PPW,), f32),
        pltpu.VMEM((PPW,), f32),
        pltpu.VMEM((PPW,), f32),
        pltpu.VMEM((RPW,), f32),
        pltpu.VMEM((RPW,), f32),
        pltpu.VMEM((RPW,), f32),
        pltpu.SemaphoreType.DMA,
        pltpu.SemaphoreType.DMA,
    ]
    run = pl.kernel(
        _sc_body, out_type=out_type, mesh=mesh,
        compiler_params=pltpu.CompilerParams(
            needs_layout_passes=False, use_tc_tiling_on_sc=False),
        scratch_types=scratch)
    oR, oG, oB = run(table, rpl, vdl, ridx_r, spl, sidx_r, lay, offb)
    return jnp.stack([oR, oG, oB], axis=1)


# speedup vs baseline: 1.3982x; 1.3982x over previous
import jax
import jax.numpy as jnp
from jax import lax
from jax.experimental import pallas as pl
from jax.experimental.pallas import tpu as pltpu
from jax.experimental.pallas import tpu_sc as plsc

RESO = 96
_R2 = RESO * RESO
_PAIR_J = RESO
_PAIR_I = 97 * RESO
N_RENDER = 262144
N_PIX = 16384
S_RAY = 16
NC, NS, L = 2, 16, 16
NW = NC * NS
PPW = N_RENDER // NW
RPW = N_PIX // NW
CH = 256
NCH = PPW // CH
NG = CH // L
_INV_R2 = 1.0 / float(_R2)
_INV_R = 1.0 / float(RESO)


def _rsqrt(x):
    xi = lax.bitcast_convert_type(x, jnp.int32)
    y = lax.bitcast_convert_type(jnp.int32(0x5F3759DF) - (xi >> 1), jnp.float32)
    h = 0.5 * x
    for _ in range(3):
        y = y * (1.5 - h * y * y)
    return y


def _decode(idxv):
    f = idxv.astype(jnp.float32)
    ii = ((f + 0.5) * _INV_R2).astype(jnp.int32)
    t = idxv - ii * _R2
    jj = ((t.astype(jnp.float32) + 0.5) * _INV_R).astype(jnp.int32)
    kk = t - jj * RESO
    return ii, jj, kk


def _sc_body(table, rin_f, ridx_r, sin_f, sidx_r, lay, offb,
             outr_h, outg_h, outb_h,
             lay_v, offb_v, fin_v, idxt_v,
             idx4_v, rows_v,
             alpha_v, rr_v, rg_v, rb_v,
             outr_v, outg_v, outb_v, semA, semB):
    wid = lax.axis_index("s") * NC + lax.axis_index("c")
    rbase = wid * RPW

    pltpu.sync_copy(lay, lay_v)
    pltpu.sync_copy(offb, offb_v)
    pltpu.sync_copy(rin_f.at[wid], fin_v)
    pltpu.sync_copy(ridx_r.at[wid], idxt_v)
    off0 = offb_v[0]
    off1 = offb_v[1]
    off2 = offb_v[2]
    off3 = offb_v[3]
    zero16 = jnp.zeros((L,), jnp.int32)
    one16 = jnp.full((L,), 1, jnp.int32)
    two16 = jnp.full((L,), 2, jnp.int32)
    lanes = lax.iota(jnp.int32, L)
    sems = (semA, semB)

    def fire(c, buf):
        for h in range(2):
            def gidx(g2, c2, _h=h):
                idxv = idxt_v[pl.ds(c * CH + _h * 128 + g2 * L, L)]
                ii, jj, kk = _decode(idxv)
                p0 = (ii * 97 + jj) * RESO + kk
                for r, off in enumerate(
                        (0, _PAIR_J, _PAIR_I, _PAIR_I + _PAIR_J)):
                    idx4_v[buf, 2 * r + _h, pl.ds(g2 * L, L)] = p0 + off
                return c2
            lax.fori_loop(0, NG // 2, gidx, 0)
        for r in range(4):
            for h in range(2):
                pltpu.async_copy(
                    table.at[idx4_v.at[buf, 2 * r + h]],
                    rows_v.at[buf, pl.ds(r * CH + h * 128, 128)], sems[buf])

    def wait_rows(buf):
        pltpu.make_async_copy(table.at[pl.ds(0, 4 * CH)],
                              rows_v.at[buf], sems[buf]).wait()

    def compute(c, buf):
        def gcomp(g, c2):
            o = c * CH + g * L
            pt = g * L + lanes
            idxv = idxt_v[pl.ds(o, L)]
            ii, jj, kk = _decode(idxv)
            xq = fin_v[0, pl.ds(o, L)]
            yq = fin_v[1, pl.ds(o, L)]
            zq = fin_v[2, pl.ds(o, L)]
            fxv = jnp.clip(xq - ii.astype(jnp.float32), 0.0, 1.0)
            fyv = jnp.clip(yq - jj.astype(jnp.float32), 0.0, 1.0)
            fzv = jnp.clip(zq - kk.astype(jnp.float32), 0.0, 1.0)

            rv = rows_v.at[buf]
            vals = []
            for ch in range(8):
                cz = []
                for q in range(4):
                    aa = plsc.load_gather(
                        rv, [pt + q * CH, jnp.full((L,), ch, jnp.int32)])
                    bb = plsc.load_gather(
                        rv, [pt + q * CH, jnp.full((L,), ch + 8, jnp.int32)])
                    cz.append(aa + (bb - aa) * fzv)
                cy0 = cz[0] + (cz[1] - cz[0]) * fyv
                cy1 = cz[2] + (cz[3] - cz[2]) * fyv
                vals.append(cy0 + (cy1 - cy0) * fxv)

            ax = plsc.load_gather(lay_v, [zero16, ii])
            ay = plsc.load_gather(lay_v, [one16, jj])
            az = plsc.load_gather(lay_v, [two16, kk])
            gxv = 2.0 * ax * xq + off0
            gyv = 2.0 * ay * yq + off1
            gzv = 2.0 * az * zq + off2
            gg = gxv * gxv + gyv * gyv + gzv * gzv
            ninv = 1.0 / (gg * _rsqrt(gg) + 1e-8)
            nx = gxv * ninv
            ny = gyv * ninv
            nz = gzv * ninv

            wx = fin_v[3, pl.ds(o, L)]
            wy = fin_v[4, pl.ds(o, L)]
            wz = fin_v[5, pl.ds(o, L)]
            vgg = wx * wx + wy * wy + wz * wz
            vinv = 1.0 / (vgg * _rsqrt(vgg) + 1e-8)
            ux = wx * vinv
            uy = wy * vinv
            uz = wz * vinv

            ndv = jnp.maximum(0.0, -(nx * ux + ny * uy + nz * uz))
            rx = ux + 2.0 * ndv * nx
            ry = uy + 2.0 * ndv * ny
            rz = uz + 2.0 * ndv * nz
            sdot = jnp.maximum(0.0, -(rx * ux + ry * uy + rz * uz))
            s2 = sdot * sdot
            s8 = (s2 * s2) * (s2 * s2)
            sk = s8 * vals[6]
            amb = 0.1 * vals[7]
            dst = pl.ds(o, L)
            rr_v[dst] = vals[0] * ndv + vals[3] * sk + amb * vals[0]
            rg_v[dst] = vals[1] * ndv + vals[4] * sk + amb * vals[1]
            rb_v[dst] = vals[2] * ndv + vals[5] * sk + amb * vals[2]
            return c2
        lax.fori_loop(0, NG, gcomp, 0)

    fire(0, 0)

    def render_pair(p, carry):
        c0 = 2 * p
        fire(c0 + 1, 1)
        wait_rows(0)
        compute(c0, 0)

        @pl.when(p < NCH // 2 - 1)
        def _():
            fire(c0 + 2, 0)
        wait_rows(1)
        compute(c0 + 1, 1)
        return carry
    lax.fori_loop(0, NCH // 2, render_pair, 0)

    pltpu.sync_copy(sin_f.at[wid], fin_v.at[pl.ds(0, 3)])
    pltpu.sync_copy(sidx_r.at[wid], idxt_v)

    def sdf_group(g, carry):
        o = g * L
        idxv = idxt_v[pl.ds(o, L)]
        ii, jj, kk = _decode(idxv)
        ax = plsc.load_gather(lay_v, [zero16, ii])
        ay = plsc.load_gather(lay_v, [one16, jj])
        az = plsc.load_gather(lay_v, [two16, kk])
        xq = fin_v[0, pl.ds(o, L)]
        yq = fin_v[1, pl.ds(o, L)]
        zq = fin_v[2, pl.ds(o, L)]
        sdfv = (ax * xq * xq + ay * yq * yq + az * zq * zq
                + off0 * xq + off1 * yq + off2 * zq + off3) / float(RESO)
        alpha_v[pl.ds(o, L)] = 1.0 / (1.0 + jnp.exp(5.0 * sdfv))
        return carry
    lax.fori_loop(0, PPW // L, sdf_group, 0)

    def ray_chunk(rc, carry):
        lidx = lanes * S_RAY + rc * (L * S_RAY)

        def step(s, st):
            T, aR, aG, aB = st
            idx = lidx + s
            av = plsc.load_gather(alpha_v, [idx])
            rv = plsc.load_gather(rr_v, [idx])
            gv = plsc.load_gather(rg_v, [idx])
            bv = plsc.load_gather(rb_v, [idx])
            w = av * T
            return (T * (1.0 - av + 1e-10),
                    aR + w * rv, aG + w * gv, aB + w * bv)
        T0 = jnp.ones((L,), jnp.float32)
        Z0 = jnp.zeros((L,), jnp.float32)
        _, aR, aG, aB = lax.fori_loop(0, S_RAY, step, (T0, Z0, Z0, Z0))
        outr_v[pl.ds(rc * L, L)] = aR
        outg_v[pl.ds(rc * L, L)] = aG
        outb_v[pl.ds(rc * L, L)] = aB
        return carry
    lax.fori_loop(0, RPW // L, ray_chunk, 0)

    pltpu.sync_copy(outr_v, outr_h.at[pl.ds(rbase, RPW)])
    pltpu.sync_copy(outg_v, outg_h.at[pl.ds(rbase, RPW)])
    pltpu.sync_copy(outb_v, outb_h.at[pl.ds(rbase, RPW)])


def kernel(renderPointList, renderIndexList, sdfPointList, sdfIndexList,
           viewDirList, rayList, xLayer, yLayer, zLayer, offset, renderData):
    del rayList
    f32 = jnp.float32
    compact = jnp.concatenate(
        [renderData[..., 0:3], renderData[..., 27:32]], axis=-1)
    pairs = jnp.concatenate(
        [compact[:, :, :RESO, :], compact[:, :, 1:, :]], axis=-1)
    table = pairs.reshape(97 * 97 * RESO, 16)

    ridx_r = renderIndexList.astype(jnp.int32).reshape(NW, PPW)
    sidx_r = sdfIndexList.astype(jnp.int32).reshape(NW, PPW)
    rp = renderPointList.astype(f32)
    vd = viewDirList.astype(f32)
    sp = sdfPointList.astype(f32)
    rin_f = jnp.stack(
        [rp[:, 0].reshape(NW, PPW), rp[:, 1].reshape(NW, PPW),
         rp[:, 2].reshape(NW, PPW), vd[:, 0].reshape(NW, PPW),
         vd[:, 1].reshape(NW, PPW), vd[:, 2].reshape(NW, PPW)], axis=1)
    sin_f = jnp.stack(
        [sp[:, 0].reshape(NW, PPW), sp[:, 1].reshape(NW, PPW),
         sp[:, 2].reshape(NW, PPW)], axis=1)
    lay = jnp.stack([xLayer, yLayer, zLayer], axis=0).astype(f32)
    offb = jnp.broadcast_to(offset.astype(f32)[:, None], (4, L))

    mesh = plsc.VectorSubcoreMesh(core_axis_name="c", subcore_axis_name="s")
    out_type = (jax.ShapeDtypeStruct((N_PIX,), f32),) * 3
    scratch = [
        pltpu.VMEM((3, RESO), f32),
        pltpu.VMEM((4, L), f32),
        pltpu.VMEM((6, PPW), f32),
        pltpu.VMEM((PPW,), jnp.int32),
        pltpu.VMEM((2, 8, 128), jnp.int32),
        pltpu.VMEM((2, 4 * CH, 16), f32),
        pltpu.VMEM((PPW,), f32),
        pltpu.VMEM((PPW,), f32),
        pltpu.VMEM((PPW,), f32),
        pltpu.VMEM((PPW,), f32),
        pltpu.VMEM((RPW,), f32),
        pltpu.VMEM((RPW,), f32),
        pltpu.VMEM((RPW,), f32),
        pltpu.SemaphoreType.DMA,
        pltpu.SemaphoreType.DMA,
    ]
    run = pl.kernel(
        _sc_body, out_type=out_type, mesh=mesh,
        compiler_params=pltpu.CompilerParams(
            needs_layout_passes=False, use_tc_tiling_on_sc=False),
        scratch_types=scratch)
    oR, oG, oB = run(table, rin_f, ridx_r, sin_f, sidx_r, lay, offb)
    return jnp.stack([oR, oG, oB], axis=1)


# speedup vs baseline: 1.4006x; 1.0018x over previous
import jax
import jax.numpy as jnp
from jax import lax
from jax.experimental import pallas as pl
from jax.experimental.pallas import tpu as pltpu
from jax.experimental.pallas import tpu_sc as plsc

RESO = 96
_R2 = RESO * RESO
_PAIR_J = RESO
_PAIR_I = 97 * RESO
N_RENDER = 262144
N_PIX = 16384
S_RAY = 16
NC, NS, L = 2, 16, 16
NW = NC * NS
PPW = N_RENDER // NW
RPW = N_PIX // NW
CH = 128
NCH = PPW // CH
NG = CH // L
_INV_R2 = 1.0 / float(_R2)
_INV_R = 1.0 / float(RESO)


def _rsqrt(x):
    xi = lax.bitcast_convert_type(x, jnp.int32)
    y = lax.bitcast_convert_type(jnp.int32(0x5F3759DF) - (xi >> 1), jnp.float32)
    h = 0.5 * x
    for _ in range(3):
        y = y * (1.5 - h * y * y)
    return y


def _decode(idxv):
    f = idxv.astype(jnp.float32)
    ii = ((f + 0.5) * _INV_R2).astype(jnp.int32)
    t = idxv - ii * _R2
    jj = ((t.astype(jnp.float32) + 0.5) * _INV_R).astype(jnp.int32)
    kk = t - jj * RESO
    return ii, jj, kk


def _sc_body(table, rin_f, ridx_r, sin_f, sidx_r, lay, offb,
             outr_h, outg_h, outb_h,
             lay_v, offb_v, fin_v, idxt_v,
             idx4_v, rows_v,
             alpha_v, rr_v, rg_v, rb_v,
             outr_v, outg_v, outb_v, semA, semB):
    wid = lax.axis_index("s") * NC + lax.axis_index("c")
    rbase = wid * RPW

    pltpu.sync_copy(lay, lay_v)
    pltpu.sync_copy(offb, offb_v)
    pltpu.sync_copy(rin_f.at[wid], fin_v)
    pltpu.sync_copy(ridx_r.at[wid], idxt_v)
    off0 = offb_v[0]
    off1 = offb_v[1]
    off2 = offb_v[2]
    off3 = offb_v[3]
    zero16 = jnp.zeros((L,), jnp.int32)
    one16 = jnp.full((L,), 1, jnp.int32)
    two16 = jnp.full((L,), 2, jnp.int32)
    lanes = lax.iota(jnp.int32, L)
    sems = (semA, semB)

    def fire(c, buf):
        def gidx(g, c2):
            idxv = idxt_v[pl.ds(c * CH + g * L, L)]
            ii, jj, kk = _decode(idxv)
            p0 = (ii * 97 + jj) * RESO + kk
            for r, off in enumerate((0, _PAIR_J, _PAIR_I, _PAIR_I + _PAIR_J)):
                idx4_v[buf, r, pl.ds(g * L, L)] = p0 + off
            return c2
        lax.fori_loop(0, NG, gidx, 0)
        for r in range(4):
            pltpu.async_copy(table.at[idx4_v.at[buf, r]],
                             rows_v.at[buf, pl.ds(r * CH, CH)], sems[buf])

    def wait_rows(buf):
        pltpu.make_async_copy(table.at[pl.ds(0, 4 * CH)],
                              rows_v.at[buf], sems[buf]).wait()

    def compute(c, buf):
        def gcomp(g, c2):
            o = c * CH + g * L
            pt = g * L + lanes
            idxv = idxt_v[pl.ds(o, L)]
            ii, jj, kk = _decode(idxv)
            xq = fin_v[0, pl.ds(o, L)]
            yq = fin_v[1, pl.ds(o, L)]
            zq = fin_v[2, pl.ds(o, L)]
            fxv = jnp.clip(xq - ii.astype(jnp.float32), 0.0, 1.0)
            fyv = jnp.clip(yq - jj.astype(jnp.float32), 0.0, 1.0)
            fzv = jnp.clip(zq - kk.astype(jnp.float32), 0.0, 1.0)

            rv = rows_v.at[buf]
            vals = []
            for ch in range(8):
                cz = []
                for q in range(4):
                    aa = plsc.load_gather(
                        rv, [pt + q * CH, jnp.full((L,), ch, jnp.int32)])
                    bb = plsc.load_gather(
                        rv, [pt + q * CH, jnp.full((L,), ch + 8, jnp.int32)])
                    cz.append(aa + (bb - aa) * fzv)
                cy0 = cz[0] + (cz[1] - cz[0]) * fyv
                cy1 = cz[2] + (cz[3] - cz[2]) * fyv
                vals.append(cy0 + (cy1 - cy0) * fxv)

            ax = plsc.load_gather(lay_v, [zero16, ii])
            ay = plsc.load_gather(lay_v, [one16, jj])
            az = plsc.load_gather(lay_v, [two16, kk])
            gxv = 2.0 * ax * xq + off0
            gyv = 2.0 * ay * yq + off1
            gzv = 2.0 * az * zq + off2
            gg = gxv * gxv + gyv * gyv + gzv * gzv
            ninv = 1.0 / (gg * _rsqrt(gg) + 1e-8)
            nx = gxv * ninv
            ny = gyv * ninv
            nz = gzv * ninv

            wx = fin_v[3, pl.ds(o, L)]
            wy = fin_v[4, pl.ds(o, L)]
            wz = fin_v[5, pl.ds(o, L)]
            vgg = wx * wx + wy * wy + wz * wz
            vinv = 1.0 / (vgg * _rsqrt(vgg) + 1e-8)
            ux = wx * vinv
            uy = wy * vinv
            uz = wz * vinv

            ndv = jnp.maximum(0.0, -(nx * ux + ny * uy + nz * uz))
            rx = ux + 2.0 * ndv * nx
            ry = uy + 2.0 * ndv * ny
            rz = uz + 2.0 * ndv * nz
            sdot = jnp.maximum(0.0, -(rx * ux + ry * uy + rz * uz))
            s2 = sdot * sdot
            s8 = (s2 * s2) * (s2 * s2)
            sk = s8 * vals[6]
            amb = 0.1 * vals[7]
            dst = pl.ds(o, L)
            rr_v[dst] = vals[0] * ndv + vals[3] * sk + amb * vals[0]
            rg_v[dst] = vals[1] * ndv + vals[4] * sk + amb * vals[1]
            rb_v[dst] = vals[2] * ndv + vals[5] * sk + amb * vals[2]
            return c2
        lax.fori_loop(0, NG, gcomp, 0)

    fire(0, 0)

    def render_pair(p, carry):
        c0 = 2 * p
        fire(c0 + 1, 1)
        wait_rows(0)
        compute(c0, 0)

        @pl.when(p < NCH // 2 - 1)
        def _():
            fire(c0 + 2, 0)
        wait_rows(1)
        compute(c0 + 1, 1)
        return carry
    lax.fori_loop(0, NCH // 2, render_pair, 0)

    pltpu.sync_copy(sin_f.at[wid], fin_v.at[pl.ds(0, 3)])
    pltpu.sync_copy(sidx_r.at[wid], idxt_v)

    def sdf_group(g, carry):
        o = g * L
        idxv = idxt_v[pl.ds(o, L)]
        ii, jj, kk = _decode(idxv)
        ax = plsc.load_gather(lay_v, [zero16, ii])
        ay = plsc.load_gather(lay_v, [one16, jj])
        az = plsc.load_gather(lay_v, [two16, kk])
        xq = fin_v[0, pl.ds(o, L)]
        yq = fin_v[1, pl.ds(o, L)]
        zq = fin_v[2, pl.ds(o, L)]
        sdfv = (ax * xq * xq + ay * yq * yq + az * zq * zq
                + off0 * xq + off1 * yq + off2 * zq + off3) / float(RESO)
        alpha_v[pl.ds(o, L)] = 1.0 / (1.0 + jnp.exp(5.0 * sdfv))
        return carry
    lax.fori_loop(0, PPW // L, sdf_group, 0)

    def ray_chunk(rc, carry):
        lidx = lanes * S_RAY + rc * (L * S_RAY)

        def step(s, st):
            T, aR, aG, aB = st
            idx = lidx + s
            av = plsc.load_gather(alpha_v, [idx])
            rv = plsc.load_gather(rr_v, [idx])
            gv = plsc.load_gather(rg_v, [idx])
            bv = plsc.load_gather(rb_v, [idx])
            w = av * T
            return (T * (1.0 - av + 1e-10),
                    aR + w * rv, aG + w * gv, aB + w * bv)
        T0 = jnp.ones((L,), jnp.float32)
        Z0 = jnp.zeros((L,), jnp.float32)
        _, aR, aG, aB = lax.fori_loop(0, S_RAY, step, (T0, Z0, Z0, Z0))
        outr_v[pl.ds(rc * L, L)] = aR
        outg_v[pl.ds(rc * L, L)] = aG
        outb_v[pl.ds(rc * L, L)] = aB
        return carry
    lax.fori_loop(0, RPW // L, ray_chunk, 0)

    pltpu.sync_copy(outr_v, outr_h.at[pl.ds(rbase, RPW)])
    pltpu.sync_copy(outg_v, outg_h.at[pl.ds(rbase, RPW)])
    pltpu.sync_copy(outb_v, outb_h.at[pl.ds(rbase, RPW)])


def kernel(renderPointList, renderIndexList, sdfPointList, sdfIndexList,
           viewDirList, rayList, xLayer, yLayer, zLayer, offset, renderData):
    del rayList
    f32 = jnp.float32
    compact = jnp.concatenate(
        [renderData[..., 0:3], renderData[..., 27:32]], axis=-1)
    pairs = jnp.concatenate(
        [compact[:, :, :RESO, :], compact[:, :, 1:, :]], axis=-1)
    table = pairs.reshape(97 * 97 * RESO, 16)

    ridx_r = renderIndexList.astype(jnp.int32).reshape(NW, PPW)
    sidx_r = sdfIndexList.astype(jnp.int32).reshape(NW, PPW)
    rp = renderPointList.astype(f32)
    vd = viewDirList.astype(f32)
    sp = sdfPointList.astype(f32)
    rin_f = jnp.stack(
        [rp[:, 0].reshape(NW, PPW), rp[:, 1].reshape(NW, PPW),
         rp[:, 2].reshape(NW, PPW), vd[:, 0].reshape(NW, PPW),
         vd[:, 1].reshape(NW, PPW), vd[:, 2].reshape(NW, PPW)], axis=1)
    sin_f = jnp.stack(
        [sp[:, 0].reshape(NW, PPW), sp[:, 1].reshape(NW, PPW),
         sp[:, 2].reshape(NW, PPW)], axis=1)
    lay = jnp.stack([xLayer, yLayer, zLayer], axis=0).astype(f32)
    offb = jnp.broadcast_to(offset.astype(f32)[:, None], (4, L))

    mesh = plsc.VectorSubcoreMesh(core_axis_name="c", subcore_axis_name="s")
    out_type = (jax.ShapeDtypeStruct((N_PIX,), f32),) * 3
    scratch = [
        pltpu.VMEM((3, RESO), f32),
        pltpu.VMEM((4, L), f32),
        pltpu.VMEM((6, PPW), f32),
        pltpu.VMEM((PPW,), jnp.int32),
        pltpu.VMEM((2, 4, CH), jnp.int32),
        pltpu.VMEM((2, 4 * CH, 16), f32),
        pltpu.VMEM((PPW,), f32),
        pltpu.VMEM((PPW,), f32),
        pltpu.VMEM((PPW,), f32),
        pltpu.VMEM((PPW,), f32),
        pltpu.VMEM((RPW,), f32),
        pltpu.VMEM((RPW,), f32),
        pltpu.VMEM((RPW,), f32),
        pltpu.SemaphoreType.DMA,
        pltpu.SemaphoreType.DMA,
    ]
    run = pl.kernel(
        _sc_body, out_type=out_type, mesh=mesh,
        compiler_params=pltpu.CompilerParams(
            needs_layout_passes=False, use_tc_tiling_on_sc=False),
        scratch_types=scratch)
    oR, oG, oB = run(table, rin_f, ridx_r, sin_f, sidx_r, lay, offb)
    return jnp.stack([oR, oG, oB], axis=1)
